# baseline (speedup 1.0000x reference)
import functools

import jax
import jax.numpy as jnp
from jax import lax
from jax.experimental import pallas as pl
from jax.experimental.pallas import tpu as pltpu
from jax.experimental.pallas import tpu_sc as plsc

B, L = 16, 1024
N = B * L
H2 = 512
HIDDEN = 2 * H2
NUM_RES = 32
N_ATOM_MASK = 9
NUM_CORES = 2
NUM_SUBCORES = 16
NW = NUM_CORES * NUM_SUBCORES
RPW = N // NW
T = 32
NCHUNK = RPW // T
LANES = 16
JV = H2 // LANES


def _body(x_hbm, bpe_hbm, mask_hbm, tok_hbm, bpe_emb_hbm, amask_hbm,
          out_hbm, idx_tok_v, idx_bpe_v, mask_v, rows_bpe_v,
          amask_v, maskrow_v, tok_v, sem_g, sem_w):
    wid = lax.axis_index("s") * NUM_CORES + lax.axis_index("c")
    base = pl.multiple_of(wid * RPW, RPW)

    cps = [
        pltpu.async_copy(x_hbm.at[pl.ds(base, RPW)], idx_tok_v, sem_g[0]),
        pltpu.async_copy(bpe_hbm.at[pl.ds(base, RPW)], idx_bpe_v, sem_g[0]),
        pltpu.async_copy(mask_hbm.at[pl.ds(base, RPW)], mask_v, sem_g[0]),
        pltpu.async_copy(tok_hbm, tok_v, sem_g[0]),
        pltpu.async_copy(amask_hbm, amask_v, sem_g[0]),
    ]
    for cp in cps:
        cp.wait()

    for j in range(JV):
        acc = amask_v[0, pl.ds(j * LANES, LANES)]
        for r in range(1, N_ATOM_MASK):
            acc = acc + amask_v[r, pl.ds(j * LANES, LANES)]
        maskrow_v[pl.ds(j * LANES, LANES)] = acc

    def issue_gather(c, b):
        off = pl.multiple_of(c * T, T)
        pltpu.async_copy(bpe_emb_hbm.at[idx_bpe_v.at[pl.ds(off, T)]],
                         rows_bpe_v.at[b], sem_g[b])

    def wait_gather(b):
        pltpu.make_async_copy(bpe_emb_hbm.at[pl.ds(0, T)],
                              rows_bpe_v.at[b], sem_g[b]).wait()

    def issue_writes(c, b):
        off = pl.multiple_of(c * T, T)
        row0 = pl.multiple_of(base + off, T)
        pltpu.async_copy(rows_bpe_v.at[b],
                         out_hbm.at[pl.ds(row0, T), pl.ds(H2, H2)], sem_w[b])
        def tok_writes(g, _):
            ivec = idx_tok_v[pl.ds(off + g * LANES, LANES)]
            r0 = row0 + g * LANES
            for t16 in range(LANES):
                pltpu.async_copy(
                    tok_v.at[pl.ds(ivec[t16], 1)],
                    out_hbm.at[pl.ds(r0 + t16, 1), pl.ds(0, H2)], sem_w[b])
            return 0

        lax.fori_loop(0, T // LANES, tok_writes, 0)

    def wait_writes(b):
        pltpu.make_async_copy(rows_bpe_v.at[b],
                              out_hbm.at[pl.ds(base, T), pl.ds(H2, H2)],
                              sem_w[b]).wait()
        pltpu.make_async_copy(rows_bpe_v.at[b],
                              out_hbm.at[pl.ds(base, T), pl.ds(0, H2)],
                              sem_w[b]).wait()

    def fix_mask(c, b):
        off = pl.multiple_of(c * T, T)

        def fix(g, _):
            mvec = mask_v[pl.ds(off + g * LANES, LANES)]
            t0 = g * LANES
            for t16 in range(LANES):
                @pl.when(mvec[t16] != 0)
                def _():
                    @pl.loop(0, JV, unroll=8)
                    def _(j):
                        rows_bpe_v[b, t0 + t16, pl.ds(j * LANES, LANES)] = (
                            maskrow_v[pl.ds(j * LANES, LANES)])
            return 0

        lax.fori_loop(0, T // LANES, fix, 0)

    issue_gather(0, 0)

    def step(i, _):
        for bb in range(2):
            c = 2 * i + bb

            @pl.when(c + 1 < NCHUNK)
            def _():
                @pl.when(c >= 1)
                def _():
                    wait_writes(1 - bb)

                issue_gather(c + 1, 1 - bb)

            wait_gather(bb)
            fix_mask(c, bb)
            issue_writes(c, bb)
        return 0

    lax.fori_loop(0, NCHUNK // 2, step, 0, unroll=False)
    wait_writes(0)
    wait_writes(1)


def _mesh_kernel():
    mesh = plsc.VectorSubcoreMesh(core_axis_name="c", subcore_axis_name="s")
    return functools.partial(
        pl.kernel,
        mesh=mesh,
        out_type=jax.ShapeDtypeStruct((N, HIDDEN), jnp.float32),
        scratch_types=[
            pltpu.VMEM((RPW,), jnp.int32),
            pltpu.VMEM((RPW,), jnp.int32),
            pltpu.VMEM((RPW,), jnp.int32),
            pltpu.VMEM((2, T, H2), jnp.float32),
            pltpu.VMEM((N_ATOM_MASK, H2), jnp.float32),
            pltpu.VMEM((H2,), jnp.float32),
            pltpu.VMEM((NUM_RES, H2), jnp.float32),
            [pltpu.SemaphoreType.DMA, pltpu.SemaphoreType.DMA],
            [pltpu.SemaphoreType.DMA, pltpu.SemaphoreType.DMA],
        ],
    )(_body)


@jax.jit
def kernel(x, bpe, mask_aa, token_embed, bpe_embed, atom_mask_embedding):
    out = _mesh_kernel()(x.reshape(N), bpe.reshape(N), mask_aa.reshape(N),
                         token_embed, bpe_embed, atom_mask_embedding)
    return out.reshape(B, L, HIDDEN)


# speedup vs baseline: 1.0883x; 1.0883x over previous
import functools

import jax
import jax.numpy as jnp
from jax import lax
from jax.experimental import pallas as pl
from jax.experimental.pallas import tpu as pltpu
from jax.experimental.pallas import tpu_sc as plsc

B, L = 16, 1024
N = B * L
H2 = 512
HIDDEN = 2 * H2
NUM_RES = 32
N_ATOM_MASK = 9
NUM_CORES = 2
NUM_SUBCORES = 16
NW = NUM_CORES * NUM_SUBCORES
RPW = N // NW
T = 64
NCHUNK = RPW // T
LANES = 16
JV = H2 // LANES


def _body(x_hbm, bpe_hbm, mask_hbm, tok_hbm, bpe_emb_hbm, amask_hbm,
          out_hbm, idx_tok_v, idx_bpe_v, mask_v, rows_bpe_v,
          amask_v, maskrow_v, tok_v, sem_g, sem_w):
    wid = lax.axis_index("s") * NUM_CORES + lax.axis_index("c")
    base = pl.multiple_of(wid * RPW, RPW)

    cps = [
        pltpu.async_copy(x_hbm.at[pl.ds(base, RPW)], idx_tok_v, sem_g[0]),
        pltpu.async_copy(bpe_hbm.at[pl.ds(base, RPW)], idx_bpe_v, sem_g[0]),
        pltpu.async_copy(mask_hbm.at[pl.ds(base, RPW)], mask_v, sem_g[0]),
        pltpu.async_copy(tok_hbm, tok_v, sem_g[0]),
        pltpu.async_copy(amask_hbm, amask_v, sem_g[0]),
    ]
    for cp in cps:
        cp.wait()

    for j in range(JV):
        acc = amask_v[0, pl.ds(j * LANES, LANES)]
        for r in range(1, N_ATOM_MASK):
            acc = acc + amask_v[r, pl.ds(j * LANES, LANES)]
        maskrow_v[pl.ds(j * LANES, LANES)] = acc

    def issue_gather(c, b):
        off = pl.multiple_of(c * T, T)
        pltpu.async_copy(bpe_emb_hbm.at[idx_bpe_v.at[pl.ds(off, T)]],
                         rows_bpe_v.at[b], sem_g[b])

    def wait_gather(b):
        pltpu.make_async_copy(bpe_emb_hbm.at[pl.ds(0, T)],
                              rows_bpe_v.at[b], sem_g[b]).wait()

    def issue_writes(c, b):
        off = pl.multiple_of(c * T, T)
        row0 = pl.multiple_of(base + off, T)
        pltpu.async_copy(rows_bpe_v.at[b],
                         out_hbm.at[pl.ds(row0, T), pl.ds(H2, H2)], sem_w[b])
        def tok_writes(g, _):
            ivec = idx_tok_v[pl.ds(off + g * LANES, LANES)]
            r0 = row0 + g * LANES
            for t16 in range(LANES):
                pltpu.async_copy(
                    tok_v.at[pl.ds(ivec[t16], 1)],
                    out_hbm.at[pl.ds(r0 + t16, 1), pl.ds(0, H2)], sem_w[b])
            return 0

        lax.fori_loop(0, T // LANES, tok_writes, 0)

    def wait_writes(b):
        pltpu.make_async_copy(rows_bpe_v.at[b],
                              out_hbm.at[pl.ds(base, T), pl.ds(H2, H2)],
                              sem_w[b]).wait()
        pltpu.make_async_copy(rows_bpe_v.at[b],
                              out_hbm.at[pl.ds(base, T), pl.ds(0, H2)],
                              sem_w[b]).wait()

    def fix_mask(c, b):
        off = pl.multiple_of(c * T, T)

        def fix(g, _):
            mvec = mask_v[pl.ds(off + g * LANES, LANES)]
            t0 = g * LANES
            for t16 in range(LANES):
                @pl.when(mvec[t16] != 0)
                def _():
                    @pl.loop(0, JV, unroll=8)
                    def _(j):
                        rows_bpe_v[b, t0 + t16, pl.ds(j * LANES, LANES)] = (
                            maskrow_v[pl.ds(j * LANES, LANES)])
            return 0

        lax.fori_loop(0, T // LANES, fix, 0)

    issue_gather(0, 0)

    def step(i, _):
        for bb in range(2):
            c = 2 * i + bb

            @pl.when(c + 1 < NCHUNK)
            def _():
                @pl.when(c >= 1)
                def _():
                    wait_writes(1 - bb)

                issue_gather(c + 1, 1 - bb)

            wait_gather(bb)
            fix_mask(c, bb)
            issue_writes(c, bb)
        return 0

    lax.fori_loop(0, NCHUNK // 2, step, 0, unroll=False)
    wait_writes(0)
    wait_writes(1)


def _mesh_kernel():
    mesh = plsc.VectorSubcoreMesh(core_axis_name="c", subcore_axis_name="s")
    return functools.partial(
        pl.kernel,
        mesh=mesh,
        out_type=jax.ShapeDtypeStruct((N, HIDDEN), jnp.float32),
        scratch_types=[
            pltpu.VMEM((RPW,), jnp.int32),
            pltpu.VMEM((RPW,), jnp.int32),
            pltpu.VMEM((RPW,), jnp.int32),
            pltpu.VMEM((2, T, H2), jnp.float32),
            pltpu.VMEM((N_ATOM_MASK, H2), jnp.float32),
            pltpu.VMEM((H2,), jnp.float32),
            pltpu.VMEM((NUM_RES, H2), jnp.float32),
            [pltpu.SemaphoreType.DMA, pltpu.SemaphoreType.DMA],
            [pltpu.SemaphoreType.DMA, pltpu.SemaphoreType.DMA],
        ],
    )(_body)


@jax.jit
def kernel(x, bpe, mask_aa, token_embed, bpe_embed, atom_mask_embedding):
    out = _mesh_kernel()(x.reshape(N), bpe.reshape(N), mask_aa.reshape(N),
                         token_embed, bpe_embed, atom_mask_embedding)
    return out.reshape(B, L, HIDDEN)
